# baseline (speedup 1.0000x reference)
import jax
import jax.numpy as jnp
from jax.experimental import pallas as pl
from jax.experimental.pallas import tpu as pltpu


def _dot_t(a, w):
    return jax.lax.dot_general(a, w, (((1,), (1,)), ((), ())),
                               preferred_element_type=jnp.float32)


def _bilstm_kernel(ids_ref,
                   emb_hbm,
                   wihf_hbm,
                   wihb_hbm,
                   whhf_hbm,
                   whhb_hbm,
                   b_ref,
                   fcw_ref,
                   out_ref,
                   x_scr,
                   wihf_ref,
                   wihb_ref,
                   whhf_ref,
                   whhb_ref,
                   copy_sems,
                   w_sem):
    TB, S, _ = x_scr.shape
    B = out_ref.shape[0]
    T = TB // B
    H = whhf_ref.shape[1]
    H4 = 4 * H
    H8 = 8 * H

    def cat2(a, b):
        return jnp.concatenate([a, b], axis=1)

    def halves(v, g):
        return cat2(v[:, g * H:(g + 1) * H], v[:, H4 + g * H:H4 + (g + 1) * H])

    def step_gates(th, c):
        i_g = 0.5 + 0.5 * halves(th, 0)
        f_g = 0.5 + 0.5 * halves(th, 1)
        g_g = halves(th, 2)
        o_g = 0.5 + 0.5 * halves(th, 3)
        c = f_g * c + i_g * g_g
        return c, o_g * jnp.tanh(c)

    NBLK = len(copy_sems)
    BLK = TB // NBLK

    w_pairs = ((wihf_hbm, wihf_ref), (wihb_hbm, wihb_ref),
               (whhf_hbm, whhf_ref), (whhb_hbm, whhb_ref))
    for hbm, vmem in w_pairs:
        pltpu.make_async_copy(hbm, vmem, w_sem).start()

    for k in range(TB):
        idx = ids_ref[k % B, k // B]
        for j in range(S):
            pltpu.make_async_copy(
                emb_hbm.at[pl.ds(idx, 1), pl.ds(j * 128, 128)],
                x_scr.at[k, pl.ds(j, 1), :],
                copy_sems.at[k // BLK]).start()

    xpf_blocks, xpb_blocks = [], []
    for bk in range(NBLK):
        rows = pl.ds(bk * BLK, BLK)
        for j in range(S):
            pltpu.make_async_copy(emb_hbm.at[pl.ds(0, BLK), pl.ds(j * 128, 128)],
                                  x_scr.at[rows, j, :], copy_sems.at[bk]).wait()
        if bk == 0:
            for hbm, vmem in w_pairs:
                pltpu.make_async_copy(hbm, vmem, w_sem).wait()
        xf = _dot_t(x_scr[rows, 0, :], wihf_ref[:, 0:128])
        xb = _dot_t(x_scr[rows, 0, :], wihb_ref[:, 0:128])
        for j in range(1, S):
            xf = xf + _dot_t(x_scr[rows, j, :], wihf_ref[:, j * 128:(j + 1) * 128])
            xb = xb + _dot_t(x_scr[rows, j, :], wihb_ref[:, j * 128:(j + 1) * 128])
        xpf_blocks.append(xf)
        xpb_blocks.append(xb)
    xpf = jnp.concatenate(xpf_blocks, axis=0)
    xpb = jnp.concatenate(xpb_blocks, axis=0)
    xp = cat2(xpf, xpb) + b_ref[...]

    lane = jax.lax.broadcasted_iota(jnp.int32, (B, H8), 1)
    bwd = lane >= H4
    gsc = jnp.where((lane // H) % 4 == 2, 1.0, 0.5).astype(jnp.float32)

    whhf = whhf_ref[...]
    whhb = whhb_ref[...]

    def xsel(t):
        xf = xp[t * B:(t + 1) * B, :]
        xb = xp[(T - 1 - t) * B:(T - t) * B, :]
        return jnp.where(bwd, xb, xf)

    th = jnp.tanh(gsc * xsel(0))
    c = (0.5 + 0.5 * halves(th, 0)) * halves(th, 2)
    h = (0.5 + 0.5 * halves(th, 3)) * jnp.tanh(c)

    for t in range(1, T):
        rec = cat2(_dot_t(h[:, 0:H], whhf), _dot_t(h[:, H:2 * H], whhb))
        th = jnp.tanh(gsc * (xsel(t) + rec))
        c, h = step_gates(th, c)

    out_ref[...] = _dot_t(h, fcw_ref[...])


def kernel(text, embedding, wih_f, whh_f, bih_f, bhh_f, wih_b, whh_b, bih_b, bhh_b, fc_w, fc_b):
    B, T = text.shape
    V, E = embedding.shape
    H = whh_f.shape[1]
    O = fc_w.shape[0]
    OP = ((O + 127) // 128) * 128
    S = E // 128
    TB = T * B
    f32 = jnp.float32

    b_all = jnp.concatenate([bih_f + bhh_f, bih_b + bhh_b])[None, :]

    grid_spec = pltpu.PrefetchScalarGridSpec(
        num_scalar_prefetch=1,
        grid=(1,),
        in_specs=[
            pl.BlockSpec(memory_space=pl.ANY),
            pl.BlockSpec(memory_space=pl.ANY),
            pl.BlockSpec(memory_space=pl.ANY),
            pl.BlockSpec(memory_space=pl.ANY),
            pl.BlockSpec(memory_space=pl.ANY),
            pl.BlockSpec((1, 8 * H), lambda i, ids: (0, 0)),
            pl.BlockSpec((OP, 2 * H), lambda i, ids: (0, 0)),
        ],
        out_specs=pl.BlockSpec((B, OP), lambda i, ids: (0, 0)),
        scratch_shapes=[
            pltpu.VMEM((TB, S, 128), f32),
            pltpu.VMEM((4 * H, E), f32),
            pltpu.VMEM((4 * H, E), f32),
            pltpu.VMEM((4 * H, H), f32),
            pltpu.VMEM((4 * H, H), f32),
            pltpu.SemaphoreType.DMA((4,)),
            pltpu.SemaphoreType.DMA,
        ],
    )
    out = pl.pallas_call(
        _bilstm_kernel,
        out_shape=jax.ShapeDtypeStruct((B, OP), f32),
        grid_spec=grid_spec,
        compiler_params=pltpu.CompilerParams(
            dimension_semantics=("arbitrary",),
            disable_bounds_checks=True,
        ),
    )(text, embedding, wih_f, wih_b, whh_f, whh_b, b_all, fc_w)

    return out[:, :O] + fc_b[None, :]


# speedup vs baseline: 1.0475x; 1.0475x over previous
import jax
import jax.numpy as jnp
from jax.experimental import pallas as pl
from jax.experimental.pallas import tpu as pltpu


def _dot_t(a, w):
    return jax.lax.dot_general(a, w, (((1,), (1,)), ((), ())),
                               preferred_element_type=jnp.float32)


def _bilstm_kernel(ids_ref,
                   emb_hbm,
                   wihf_ref,
                   wihb_ref,
                   whhf_ref,
                   whhb_ref,
                   b_ref,
                   fcw_ref,
                   out_ref,
                   x_scr,
                   copy_sems):
    TB, S, _ = x_scr.shape
    B = out_ref.shape[0]
    T = TB // B
    H = whhf_ref.shape[1]
    H4 = 4 * H
    H8 = 8 * H

    def cat2(a, b):
        return jnp.concatenate([a, b], axis=1)

    def halves(v, g):
        return cat2(v[:, g * H:(g + 1) * H], v[:, H4 + g * H:H4 + (g + 1) * H])

    def step_gates(th, c):
        i_g = 0.5 + 0.5 * halves(th, 0)
        f_g = 0.5 + 0.5 * halves(th, 1)
        g_g = halves(th, 2)
        o_g = 0.5 + 0.5 * halves(th, 3)
        c = f_g * c + i_g * g_g
        return c, o_g * jnp.tanh(c)

    NBLK = len(copy_sems)
    BLK = TB // NBLK
    for k in range(TB):
        idx = ids_ref[k % B, k // B]
        for j in range(S):
            pltpu.make_async_copy(
                emb_hbm.at[pl.ds(idx, 1), pl.ds(j * 128, 128)],
                x_scr.at[k, pl.ds(j, 1), :],
                copy_sems.at[k // BLK]).start()

    xpf_blocks, xpb_blocks = [], []
    for bk in range(NBLK):
        rows = pl.ds(bk * BLK, BLK)
        for j in range(S):
            pltpu.make_async_copy(emb_hbm.at[pl.ds(0, BLK), pl.ds(j * 128, 128)],
                                  x_scr.at[rows, j, :], copy_sems.at[bk]).wait()
        xf = _dot_t(x_scr[rows, 0, :], wihf_ref[:, 0:128])
        xb = _dot_t(x_scr[rows, 0, :], wihb_ref[:, 0:128])
        for j in range(1, S):
            xf = xf + _dot_t(x_scr[rows, j, :], wihf_ref[:, j * 128:(j + 1) * 128])
            xb = xb + _dot_t(x_scr[rows, j, :], wihb_ref[:, j * 128:(j + 1) * 128])
        xpf_blocks.append(xf)
        xpb_blocks.append(xb)
    xpf = jnp.concatenate(xpf_blocks, axis=0)
    xpb = jnp.concatenate(xpb_blocks, axis=0)
    xp = cat2(xpf, xpb) + b_ref[...]

    lane = jax.lax.broadcasted_iota(jnp.int32, (B, H8), 1)
    bwd = lane >= H4
    gsc = jnp.where((lane // H) % 4 == 2, 1.0, 0.5).astype(jnp.float32)

    whhf = whhf_ref[...].astype(jnp.bfloat16)
    whhb = whhb_ref[...].astype(jnp.bfloat16)

    def xsel(t):
        xf = xp[t * B:(t + 1) * B, :]
        xb = xp[(T - 1 - t) * B:(T - t) * B, :]
        return jnp.where(bwd, xb, xf)

    th = jnp.tanh(gsc * xsel(0))
    c = (0.5 + 0.5 * halves(th, 0)) * halves(th, 2)
    h = (0.5 + 0.5 * halves(th, 3)) * jnp.tanh(c)

    for t in range(1, T):
        hb16 = h.astype(jnp.bfloat16)
        rec = cat2(_dot_t(hb16[:, 0:H], whhf), _dot_t(hb16[:, H:2 * H], whhb))
        th = jnp.tanh(gsc * (xsel(t) + rec))
        c, h = step_gates(th, c)

    out_ref[...] = _dot_t(h, fcw_ref[...])


def kernel(text, embedding, wih_f, whh_f, bih_f, bhh_f, wih_b, whh_b, bih_b, bhh_b, fc_w, fc_b):
    B, T = text.shape
    V, E = embedding.shape
    H = whh_f.shape[1]
    O = fc_w.shape[0]
    OP = ((O + 127) // 128) * 128
    S = E // 128
    TB = T * B
    f32 = jnp.float32

    b_all = jnp.concatenate([bih_f + bhh_f, bih_b + bhh_b])[None, :]

    grid_spec = pltpu.PrefetchScalarGridSpec(
        num_scalar_prefetch=1,
        grid=(1,),
        in_specs=[
            pl.BlockSpec(memory_space=pl.ANY),
            pl.BlockSpec((4 * H, E), lambda i, ids: (0, 0)),
            pl.BlockSpec((4 * H, E), lambda i, ids: (0, 0)),
            pl.BlockSpec((4 * H, H), lambda i, ids: (0, 0)),
            pl.BlockSpec((4 * H, H), lambda i, ids: (0, 0)),
            pl.BlockSpec((1, 8 * H), lambda i, ids: (0, 0)),
            pl.BlockSpec((OP, 2 * H), lambda i, ids: (0, 0)),
        ],
        out_specs=pl.BlockSpec((B, OP), lambda i, ids: (0, 0)),
        scratch_shapes=[
            pltpu.VMEM((TB, S, 128), f32),
            pltpu.SemaphoreType.DMA((4,)),
        ],
    )
    out = pl.pallas_call(
        _bilstm_kernel,
        out_shape=jax.ShapeDtypeStruct((B, OP), f32),
        grid_spec=grid_spec,
        compiler_params=pltpu.CompilerParams(
            dimension_semantics=("arbitrary",),
            disable_bounds_checks=True,
        ),
    )(text, embedding, wih_f, wih_b, whh_f, whh_b, b_all, fc_w)

    return out[:, :O] + fc_b[None, :]


# speedup vs baseline: 1.0588x; 1.0107x over previous
import jax
import jax.numpy as jnp
from jax.experimental import pallas as pl
from jax.experimental.pallas import tpu as pltpu


def _dot_t(a, w):
    return jax.lax.dot_general(a, w, (((1,), (1,)), ((), ())),
                               preferred_element_type=jnp.float32)


def _bilstm_kernel(ids_ref,
                   emb_hbm,
                   wihf_ref,
                   wihb_ref,
                   whhf_ref,
                   whhb_ref,
                   b_ref,
                   fcw_ref,
                   out_ref,
                   x_scr,
                   copy_sems):
    TB, S, _ = x_scr.shape
    B = out_ref.shape[0]
    T = TB // B
    H = whhf_ref.shape[1]
    H4 = 4 * H
    H8 = 8 * H

    def cat2(a, b):
        return jnp.concatenate([a, b], axis=1)

    def halves(v, g):
        return cat2(v[:, g * H:(g + 1) * H], v[:, H4 + g * H:H4 + (g + 1) * H])

    def step_gates(th, c):
        i_g = 0.5 + 0.5 * halves(th, 0)
        f_g = 0.5 + 0.5 * halves(th, 1)
        g_g = halves(th, 2)
        o_g = 0.5 + 0.5 * halves(th, 3)
        c = f_g * c + i_g * g_g
        return c, o_g * jnp.tanh(c)

    NBLK = len(copy_sems)
    BLK = TB // NBLK
    for k in range(TB):
        idx = ids_ref[k % B, k // B]
        for j in range(S):
            pltpu.make_async_copy(
                emb_hbm.at[pl.ds(idx, 1), pl.ds(j * 128, 128)],
                x_scr.at[k, pl.ds(j, 1), :],
                copy_sems.at[k // BLK]).start()

    xpf_blocks, xpb_blocks = [], []
    for bk in range(NBLK):
        rows = pl.ds(bk * BLK, BLK)
        for j in range(S):
            pltpu.make_async_copy(emb_hbm.at[pl.ds(0, BLK), pl.ds(j * 128, 128)],
                                  x_scr.at[rows, j, :], copy_sems.at[bk]).wait()
        xf = _dot_t(x_scr[rows, 0, :], wihf_ref[:, 0:128])
        xb = _dot_t(x_scr[rows, 0, :], wihb_ref[:, 0:128])
        for j in range(1, S):
            xf = xf + _dot_t(x_scr[rows, j, :], wihf_ref[:, j * 128:(j + 1) * 128])
            xb = xb + _dot_t(x_scr[rows, j, :], wihb_ref[:, j * 128:(j + 1) * 128])
        xpf_blocks.append(xf)
        xpb_blocks.append(xb)
    xpf = jnp.concatenate(xpf_blocks, axis=0)
    xpb = jnp.concatenate(xpb_blocks, axis=0)
    xp = cat2(xpf, xpb) + b_ref[...]

    lane = jax.lax.broadcasted_iota(jnp.int32, (B, H8), 1)
    bwd = lane >= H4
    gsc = jnp.where((lane // H) % 4 == 2, 1.0, 0.5).astype(jnp.float32)

    whhf = whhf_ref[...]
    whhb = whhb_ref[...]

    def xsel(t):
        xf = xp[t * B:(t + 1) * B, :]
        xb = xp[(T - 1 - t) * B:(T - t) * B, :]
        return jnp.where(bwd, xb, xf)

    th = jnp.tanh(gsc * xsel(0))
    c = (0.5 + 0.5 * halves(th, 0)) * halves(th, 2)
    h = (0.5 + 0.5 * halves(th, 3)) * jnp.tanh(c)

    for t in range(1, T):
        rec = cat2(_dot_t(h[:, 0:H], whhf), _dot_t(h[:, H:2 * H], whhb))
        th = jnp.tanh(gsc * (xsel(t) + rec))
        c, h = step_gates(th, c)

    out_ref[...] = _dot_t(h, fcw_ref[...])


def kernel(text, embedding, wih_f, whh_f, bih_f, bhh_f, wih_b, whh_b, bih_b, bhh_b, fc_w, fc_b):
    B, T = text.shape
    V, E = embedding.shape
    H = whh_f.shape[1]
    O = fc_w.shape[0]
    OP = ((O + 127) // 128) * 128
    S = E // 128
    TB = T * B
    f32 = jnp.float32

    b_all = jnp.concatenate([bih_f + bhh_f, bih_b + bhh_b])[None, :]

    grid_spec = pltpu.PrefetchScalarGridSpec(
        num_scalar_prefetch=1,
        grid=(1,),
        in_specs=[
            pl.BlockSpec(memory_space=pl.ANY),
            pl.BlockSpec((4 * H, E), lambda i, ids: (0, 0)),
            pl.BlockSpec((4 * H, E), lambda i, ids: (0, 0)),
            pl.BlockSpec((4 * H, H), lambda i, ids: (0, 0)),
            pl.BlockSpec((4 * H, H), lambda i, ids: (0, 0)),
            pl.BlockSpec((1, 8 * H), lambda i, ids: (0, 0)),
            pl.BlockSpec((OP, 2 * H), lambda i, ids: (0, 0)),
        ],
        out_specs=pl.BlockSpec((B, OP), lambda i, ids: (0, 0)),
        scratch_shapes=[
            pltpu.VMEM((TB, S, 128), f32),
            pltpu.SemaphoreType.DMA((8,)),
        ],
    )
    out = pl.pallas_call(
        _bilstm_kernel,
        out_shape=jax.ShapeDtypeStruct((B, OP), f32),
        grid_spec=grid_spec,
        compiler_params=pltpu.CompilerParams(
            dimension_semantics=("arbitrary",),
            disable_bounds_checks=True,
        ),
    )(text, embedding, wih_f, wih_b, whh_f, whh_b, b_all, fc_w)

    return out[:, :O] + fc_b[None, :]
